# Initial kernel scaffold; baseline (speedup 1.0000x reference)
import jax
import jax.numpy as jnp
from jax.experimental import pallas as pl


def kernel(x_mol, edge_index_mol, mol_batch, x_prot, edge_index_prot, prot_batch, params):
    raise NotImplementedError("write your pallas kernel here")


# speedup vs baseline: 2.7976x; 2.7976x over previous
import functools
import math

import jax
import jax.numpy as jnp
from jax import lax
from jax.experimental import pallas as pl
from jax.experimental.pallas import tpu as pltpu
from jax.experimental.pallas import tpu_sc as plsc

N = 4096
E = 65536
D = 128
NG = 64
HEADS = 4
HD = D // HEADS

NC = 2
NS = 16
NW = NC * NS
EPW = E // NW
CHUNK = 128
NCHUNK = EPW // CHUNK
RPT = N // NS


@functools.cache
def _make_sc_segsum(with_cnt):
    mesh = plsc.VectorSubcoreMesh(core_axis_name="c", subcore_axis_name="s",
                                  num_cores=NC, num_subcores=NS)
    fshape = jax.ShapeDtypeStruct((NC * N, D), jnp.float32)
    out_type = (fshape, fshape) if with_cnt else fshape
    scratch = [
        pltpu.VMEM((CHUNK,), jnp.int32),
        pltpu.VMEM((CHUNK,), jnp.int32),
        pltpu.VMEM((CHUNK, D), jnp.float32),
        pltpu.VMEM_SHARED((N, D), jnp.float32),
    ]
    if with_cnt:
        scratch += [
            pltpu.VMEM((CHUNK, D), jnp.float32),
            pltpu.VMEM_SHARED((N, D), jnp.float32),
        ]
    scratch.append(pltpu.SemaphoreType.DMA)

    @functools.partial(pl.kernel, out_type=out_type, mesh=mesh,
                       scratch_types=scratch)
    def sc_segsum(table_hbm, src_hbm, dst_hbm, zeros_hbm, *rest):
        if with_cnt:
            ones_hbm, out_hbm, cnt_hbm, sidx, didx, rows, acc, ones_v, acc_cnt, sem = rest
        else:
            out_hbm, sidx, didx, rows, acc, sem = rest
        cid = lax.axis_index("c")
        sid = lax.axis_index("s")
        wid = sid * NC + cid
        r0 = sid * RPT
        pltpu.sync_copy(zeros_hbm.at[pl.ds(r0, RPT)], acc.at[pl.ds(r0, RPT)])
        if with_cnt:
            pltpu.sync_copy(zeros_hbm.at[pl.ds(r0, RPT)],
                            acc_cnt.at[pl.ds(r0, RPT)])
            pltpu.sync_copy(ones_hbm, ones_v)
        plsc.subcore_barrier()

        def body(j, carry):
            base = pl.multiple_of(wid * EPW + j * CHUNK, CHUNK)
            pltpu.sync_copy(src_hbm.at[pl.ds(base, CHUNK)], sidx)
            pltpu.sync_copy(dst_hbm.at[pl.ds(base, CHUNK)], didx)
            pltpu.async_copy(table_hbm.at[sidx], rows, sem).wait()
            pltpu.sync_copy(rows, acc.at[didx], add=True)
            if with_cnt:
                pltpu.sync_copy(ones_v, acc_cnt.at[didx], add=True)
            return carry

        lax.fori_loop(0, NCHUNK, body, 0)
        plsc.subcore_barrier()
        out_base = pl.multiple_of(cid * N + r0, RPT)
        pltpu.sync_copy(acc.at[pl.ds(r0, RPT)], out_hbm.at[pl.ds(out_base, RPT)])
        if with_cnt:
            pltpu.sync_copy(acc_cnt.at[pl.ds(r0, RPT)],
                            cnt_hbm.at[pl.ds(out_base, RPT)])

    return sc_segsum


def _sc_segsum(table, src, dst, zeros, ones=None):
    if ones is None:
        return _make_sc_segsum(False)(table, src, dst, zeros)
    return _make_sc_segsum(True)(table, src, dst, zeros, ones)


def _dotT(a, w):
    return lax.dot_general(a, w, (((1,), (1,)), ((), ())),
                           preferred_element_type=jnp.float32)


BR = 512


def _sage_tc(parts, cnt_parts, x, wl, bl, wr):
    def body(p_ref, c_ref, x_ref, wl_ref, bl_ref, wr_ref, o_ref):
        s = p_ref[0] + p_ref[1]
        cnt = jnp.maximum(c_ref[0][:, :1] + c_ref[1][:, :1], 1.0)
        mean = s / cnt
        h = _dotT(mean, wl_ref[...]) + bl_ref[...] + _dotT(x_ref[...], wr_ref[...])
        o_ref[...] = jnp.maximum(h, 0.0)

    part_spec = pl.BlockSpec((NC, BR, D), lambda i: (0, i, 0))
    return pl.pallas_call(
        body,
        grid=(N // BR,),
        in_specs=[
            part_spec,
            part_spec,
            pl.BlockSpec((BR, D), lambda i: (i, 0)),
            pl.BlockSpec((D, D), lambda i: (0, 0)),
            pl.BlockSpec((1, D), lambda i: (0, 0)),
            pl.BlockSpec((D, D), lambda i: (0, 0)),
        ],
        out_specs=pl.BlockSpec((BR, D), lambda i: (i, 0)),
        out_shape=jax.ShapeDtypeStruct((N, D), jnp.float32),
    )(parts.reshape(NC, N, D), cnt_parts.reshape(NC, N, D), x, wl, bl, wr)


def _proj_tc(hm, hp, ws, bs):
    def body(hm_ref, hp_ref, w0, w1, w2, w3, w4, w5, b0, b1, b2, b3, b4, b5,
             qm, kp, vp, qp, km, vm):
        m = hm_ref[...]
        p = hp_ref[...]
        qm[...] = _dotT(m, w0[...]) + b0[...]
        kp[...] = _dotT(p, w1[...]) + b1[...]
        vp[...] = _dotT(p, w2[...]) + b2[...]
        qp[...] = _dotT(p, w3[...]) + b3[...]
        km[...] = _dotT(m, w4[...]) + b4[...]
        vm[...] = _dotT(m, w5[...]) + b5[...]

    row_spec = pl.BlockSpec((BR, D), lambda i: (i, 0))
    w_spec = pl.BlockSpec((D, D), lambda i: (0, 0))
    b_spec = pl.BlockSpec((1, D), lambda i: (0, 0))
    return pl.pallas_call(
        body,
        grid=(N // BR,),
        in_specs=[row_spec, row_spec] + [w_spec] * 6 + [b_spec] * 6,
        out_specs=[row_spec] * 6,
        out_shape=[jax.ShapeDtypeStruct((N, D), jnp.float32)] * 6,
    )(hm, hp, *ws, *bs)


BQ = 256


def _attn_tc(q, k, v, res):
    scale = 1.0 / math.sqrt(HD)

    def body(q_ref, k_ref, v_ref, r_ref, o_ref):
        kk = k_ref[...]
        vv = v_ref[...]
        outs = []
        for h in range(HEADS):
            qh = q_ref[:, h * HD:(h + 1) * HD]
            kh = kk[:, h * HD:(h + 1) * HD]
            vh = vv[:, h * HD:(h + 1) * HD]
            s = _dotT(qh, kh) * scale
            m = jnp.max(s, axis=1, keepdims=True)
            e = jnp.exp(s - m)
            w = e / jnp.sum(e, axis=1, keepdims=True)
            outs.append(lax.dot_general(w, vh, (((1,), (0,)), ((), ())),
                                        preferred_element_type=jnp.float32))
        o_ref[...] = r_ref[...] + jnp.concatenate(outs, axis=1)

    qspec = pl.BlockSpec((BQ, D), lambda i: (i, 0))
    full = pl.BlockSpec((N, D), lambda i: (0, 0))
    return pl.pallas_call(
        body,
        grid=(N // BQ,),
        in_specs=[qspec, full, full, qspec],
        out_specs=qspec,
        out_shape=jax.ShapeDtypeStruct((N, D), jnp.float32),
    )(q, k, v, res)


def _pool_tc(hm, hp, mb, pb, fc1w, fc1b, fc2w, fc2b):
    def body(hm_ref, hp_ref, mb_ref, pb_ref, w1, b1, w2, b2, o_ref):
        ones_rows = jnp.ones((N, D), jnp.float32)

        def pool(h_ref, seg_ref):
            seg = seg_ref[...]
            gids = lax.broadcasted_iota(jnp.int32, (N, NG), 1)
            onehot = (seg == gids).astype(jnp.float32)
            tot = lax.dot_general(onehot, h_ref[...], (((0,), (0,)), ((), ())),
                                  preferred_element_type=jnp.float32)
            cnt = lax.dot_general(onehot, ones_rows, (((0,), (0,)), ((), ())),
                                  preferred_element_type=jnp.float32)
            return tot / jnp.maximum(cnt, 1.0)

        z = jnp.concatenate([pool(hm_ref, mb_ref), pool(hp_ref, pb_ref)], axis=1)
        x = jnp.maximum(_dotT(z, w1[...]) + b1[...], 0.0)
        y = jnp.sum(x * w2[...], axis=1, keepdims=True) + b2[0, 0]
        o_ref[...] = 1.0 / (1.0 + jnp.exp(-y))

    full = lambda shp: pl.BlockSpec(shp, lambda: tuple(0 for _ in shp))
    return pl.pallas_call(
        body,
        in_specs=[full((N, D)), full((N, D)), full((N, 1)), full((N, 1)),
                  full((D, 2 * D)), full((1, D)), full((1, D)), full((1, 1))],
        out_specs=full((NG, 1)),
        out_shape=jax.ShapeDtypeStruct((NG, 1), jnp.float32),
    )(hm, hp, mb, pb, fc1w, fc1b, fc2w, fc2b)


def _graph_tower(x, src, dst, zeros, ones, p, g):
    parts1, cnt_parts = _sc_segsum(x, src, dst, zeros, ones)
    h1 = _sage_tc(parts1, cnt_parts, x, p[g + '_Wl1'],
                  p[g + '_bl1'].reshape(1, D), p[g + '_Wr1'])
    parts2 = _sc_segsum(h1, src, dst, zeros)
    h2 = _sage_tc(parts2, cnt_parts, h1, p[g + '_Wl2'],
                  p[g + '_bl2'].reshape(1, D), p[g + '_Wr2'])
    return h2


def kernel(x_mol, edge_index_mol, mol_batch, x_prot, edge_index_prot,
           prot_batch, params):
    p = params
    zeros = jnp.zeros((N, D), jnp.float32)
    ones = jnp.ones((CHUNK, D), jnp.float32)

    h_mol = _graph_tower(x_mol, edge_index_mol[0], edge_index_mol[1],
                         zeros, ones, p, 'mol')
    h_prot = _graph_tower(x_prot, edge_index_prot[0], edge_index_prot[1],
                          zeros, ones, p, 'prot')

    ws = [p['m2p_WQ'], p['m2p_WK'], p['m2p_WV'],
          p['p2m_WQ'], p['p2m_WK'], p['p2m_WV']]
    bs = [p['m2p_bQ'], p['m2p_bK'], p['m2p_bV'],
          p['p2m_bQ'], p['p2m_bK'], p['p2m_bV']]
    bs = [b.reshape(1, D) for b in bs]
    qm, kp, vp, qp, km, vm = _proj_tc(h_mol, h_prot, ws, bs)

    hm_c = _attn_tc(qm, kp, vp, h_mol)
    hp_c = _attn_tc(qp, km, vm, h_prot)

    out = _pool_tc(hm_c, hp_c,
                   mol_batch.reshape(N, 1), prot_batch.reshape(N, 1),
                   p['fc1_W'], p['fc1_b'].reshape(1, D),
                   p['fc2_W'].reshape(1, D), p['fc2_b'].reshape(1, 1))
    return out.reshape(NG)


# speedup vs baseline: 3.3569x; 1.1999x over previous
import functools
import math

import jax
import jax.numpy as jnp
from jax import lax
from jax.experimental import pallas as pl
from jax.experimental.pallas import tpu as pltpu
from jax.experimental.pallas import tpu_sc as plsc

N = 4096
E = 65536
D = 128
NG = 64
HEADS = 4
HD = D // HEADS

N2 = 2 * N
E2 = 2 * E

NC = 2
NS = 16
NW = NC * NS
EPW = E2 // NW
CHUNK = 128
NCH = EPW // CHUNK
NPAIR = NCH // 2
RPT = N2 // NS
CW = 128


def _mesh():
    return plsc.VectorSubcoreMesh(core_axis_name="c", subcore_axis_name="s",
                                  num_cores=NC, num_subcores=NS)


@functools.cache
def _make_sc_segsum():
    @functools.partial(
        pl.kernel,
        out_type=jax.ShapeDtypeStruct((NC * N2, D), jnp.float32),
        mesh=_mesh(),
        scratch_types=[
            pltpu.VMEM((NCH, CHUNK), jnp.int32),
            pltpu.VMEM((NCH, CHUNK), jnp.int32),
            pltpu.VMEM((CHUNK, D), jnp.float32),
            pltpu.VMEM((CHUNK, D), jnp.float32),
            pltpu.VMEM_SHARED((N2, D), jnp.float32),
            pltpu.SemaphoreType.DMA,
            pltpu.SemaphoreType.DMA,
            pltpu.SemaphoreType.DMA,
            pltpu.SemaphoreType.DMA,
        ],
    )
    def sc_segsum(table_hbm, src_hbm, dst_hbm, zeros_hbm, out_hbm,
                  sidx, didx, buf0, buf1, acc, g0, g1, s0, s1):
        cid = lax.axis_index("c")
        sid = lax.axis_index("s")
        wid = sid * NC + cid
        r0 = sid * RPT
        pltpu.sync_copy(zeros_hbm.at[pl.ds(r0, RPT)], acc.at[pl.ds(r0, RPT)])
        row0 = pl.multiple_of(wid * NCH, NCH)
        pltpu.sync_copy(src_hbm.at[pl.ds(row0, NCH)], sidx)
        pltpu.sync_copy(dst_hbm.at[pl.ds(row0, NCH)], didx)
        plsc.subcore_barrier()

        bufs = (buf0, buf1)
        gsems = (g0, g1)
        ssems = (s0, s1)

        def start_g(j, b):
            pltpu.async_copy(table_hbm.at[sidx.at[j]], bufs[b], gsems[b])

        def wait_g(b):
            pltpu.make_async_copy(table_hbm.at[sidx.at[0]], bufs[b],
                                  gsems[b]).wait()

        def start_s(j, b):
            pltpu.async_copy(bufs[b], acc.at[didx.at[j]], ssems[b], add=True)

        def wait_s(b):
            pltpu.make_async_copy(bufs[b], acc.at[didx.at[0]],
                                  ssems[b]).wait()

        start_g(0, 0)

        def pair(jj, carry):
            j0 = 2 * jj
            @pl.when(jj > 0)
            def _():
                wait_s(1)

            start_g(j0 + 1, 1)
            wait_g(0)
            start_s(j0, 0)

            @pl.when(jj < NPAIR - 1)
            def _():
                wait_s(0)
                start_g(j0 + 2, 0)

            wait_g(1)
            start_s(j0 + 1, 1)
            return carry

        lax.fori_loop(0, NPAIR, pair, 0)
        wait_s(0)
        wait_s(1)
        plsc.subcore_barrier()
        out_base = pl.multiple_of(cid * N2 + r0, RPT)
        pltpu.sync_copy(acc.at[pl.ds(r0, RPT)], out_hbm.at[pl.ds(out_base, RPT)])

    return sc_segsum


@functools.cache
def _make_sc_hist():
    @functools.partial(
        pl.kernel,
        out_type=jax.ShapeDtypeStruct((NC * N2, CW), jnp.float32),
        mesh=_mesh(),
        scratch_types=[
            pltpu.VMEM((NCH, CHUNK), jnp.int32),
            pltpu.VMEM((CHUNK, CW), jnp.float32),
            pltpu.VMEM_SHARED((N2, CW), jnp.float32),
            pltpu.SemaphoreType.DMA,
            pltpu.SemaphoreType.DMA,
        ],
    )
    def sc_hist(dst_hbm, zeros_hbm, ones_hbm, out_hbm, didx, ones_v, acc,
                s0, s1):
        cid = lax.axis_index("c")
        sid = lax.axis_index("s")
        wid = sid * NC + cid
        r0 = sid * RPT
        pltpu.sync_copy(zeros_hbm.at[pl.ds(r0, RPT)], acc.at[pl.ds(r0, RPT)])
        row0 = pl.multiple_of(wid * NCH, NCH)
        pltpu.sync_copy(dst_hbm.at[pl.ds(row0, NCH)], didx)
        pltpu.sync_copy(ones_hbm, ones_v)
        plsc.subcore_barrier()

        sems = (s0, s1)

        def fire(j, b):
            pltpu.async_copy(ones_v, acc.at[didx.at[j]], sems[b], add=True)

        def drain(b):
            pltpu.make_async_copy(ones_v, acc.at[didx.at[0]], sems[b]).wait()

        def pair(jj, carry):
            j0 = 2 * jj

            @pl.when(jj > 0)
            def _():
                drain(0)

            fire(j0, 0)

            @pl.when(jj > 0)
            def _():
                drain(1)

            fire(j0 + 1, 1)
            return carry

        lax.fori_loop(0, NPAIR, pair, 0)
        drain(0)
        drain(1)
        plsc.subcore_barrier()
        out_base = pl.multiple_of(cid * N2 + r0, RPT)
        pltpu.sync_copy(acc.at[pl.ds(r0, RPT)], out_hbm.at[pl.ds(out_base, RPT)])

    return sc_hist


def _dotT(a, w):
    return lax.dot_general(a, w, (((1,), (1,)), ((), ())),
                           preferred_element_type=jnp.float32)


BR = 512
NBLK = N2 // BR


def _cnt_full(c_ref):
    cnt = jnp.maximum(c_ref[0] + c_ref[1], 1.0)
    return jnp.concatenate([cnt] * (D // CW), axis=1)


def _sage1_tc(parts, cnt_parts, x, wl, bl, wr):
    def body(p_ref, c_ref, x_ref, wl_ref, bl_ref, wr_ref, o_ref):
        mean = (p_ref[0] + p_ref[1]) / _cnt_full(c_ref)
        h = (_dotT(mean, wl_ref[0]) + bl_ref[0]
             + _dotT(x_ref[...], wr_ref[0]))
        o_ref[...] = jnp.maximum(h, 0.0)

    return pl.pallas_call(
        body,
        grid=(NBLK,),
        in_specs=[
            pl.BlockSpec((NC, BR, D), lambda i: (0, i, 0)),
            pl.BlockSpec((NC, BR, CW), lambda i: (0, i, 0)),
            pl.BlockSpec((BR, D), lambda i: (i, 0)),
            pl.BlockSpec((1, D, D), lambda i: (i // (NBLK // 2), 0, 0)),
            pl.BlockSpec((1, 1, D), lambda i: (i // (NBLK // 2), 0, 0)),
            pl.BlockSpec((1, D, D), lambda i: (i // (NBLK // 2), 0, 0)),
        ],
        out_specs=pl.BlockSpec((BR, D), lambda i: (i, 0)),
        out_shape=jax.ShapeDtypeStruct((N2, D), jnp.float32),
    )(parts.reshape(NC, N2, D), cnt_parts.reshape(NC, N2, CW), x, wl, bl, wr)


def _sage2_proj_tc(parts, cnt_parts, x, wl, bl, wr, wq, bq, wk, bk, wv, bv):
    def body(p_ref, c_ref, x_ref, wl_ref, bl_ref, wr_ref,
             wq_ref, bq_ref, wk_ref, bk_ref, wv_ref, bv_ref,
             h_ref, q_ref, k_ref, v_ref):
        mean = (p_ref[0] + p_ref[1]) / _cnt_full(c_ref)
        h = (_dotT(mean, wl_ref[0]) + bl_ref[0]
             + _dotT(x_ref[...], wr_ref[0]))
        h = jnp.maximum(h, 0.0)
        h_ref[...] = h
        q_ref[...] = _dotT(h, wq_ref[0]) + bq_ref[0]
        k_ref[...] = _dotT(h, wk_ref[0]) + bk_ref[0]
        v_ref[...] = _dotT(h, wv_ref[0]) + bv_ref[0]

    half = NBLK // 2
    wsel = lambda i: (i // half, 0, 0)
    row = pl.BlockSpec((BR, D), lambda i: (i, 0))
    xrow = pl.BlockSpec((BR, D), lambda i: ((i + half) % NBLK, 0))
    w_spec = pl.BlockSpec((1, D, D), wsel)
    b_spec = pl.BlockSpec((1, 1, D), wsel)
    sds = jax.ShapeDtypeStruct((N2, D), jnp.float32)
    return pl.pallas_call(
        body,
        grid=(NBLK,),
        in_specs=[
            pl.BlockSpec((NC, BR, D), lambda i: (0, i, 0)),
            pl.BlockSpec((NC, BR, CW), lambda i: (0, i, 0)),
            row, w_spec, b_spec, w_spec,
            w_spec, b_spec, w_spec, b_spec, w_spec, b_spec,
        ],
        out_specs=[row, row, xrow, xrow],
        out_shape=[sds, sds, sds, sds],
    )(parts.reshape(NC, N2, D), cnt_parts.reshape(NC, N2, CW), x,
      wl, bl, wr, wq, bq, wk, bk, wv, bv)


BQ = 256


def _attn_tc(q, k, v, res):
    scale = 1.0 / math.sqrt(HD)
    qpg = N // BQ

    def body(q_ref, k_ref, v_ref, r_ref, o_ref):
        kk = k_ref[0]
        vv = v_ref[0]
        outs = []
        for h in range(HEADS):
            qh = q_ref[:, h * HD:(h + 1) * HD]
            kh = kk[:, h * HD:(h + 1) * HD]
            vh = vv[:, h * HD:(h + 1) * HD]
            s = _dotT(qh, kh) * scale
            m = jnp.max(s, axis=1, keepdims=True)
            e = jnp.exp(s - m)
            w = e / jnp.sum(e, axis=1, keepdims=True)
            outs.append(lax.dot_general(w, vh, (((1,), (0,)), ((), ())),
                                        preferred_element_type=jnp.float32))
        o_ref[...] = r_ref[...] + jnp.concatenate(outs, axis=1)

    qspec = pl.BlockSpec((BQ, D), lambda i: (i, 0))
    kv = pl.BlockSpec((1, N, D), lambda i: (i // qpg, 0, 0))
    return pl.pallas_call(
        body,
        grid=(N2 // BQ,),
        in_specs=[qspec, kv, kv, qspec],
        out_specs=qspec,
        out_shape=jax.ShapeDtypeStruct((N2, D), jnp.float32),
    )(q, k.reshape(2, N, D), v.reshape(2, N, D), res)


def _pool_tc(h_all, mb, pb, fc1w, fc1b, fc2w, fc2b):
    def body(hm_ref, hp_ref, mb_ref, pb_ref, w1, b1, w2, b2, o_ref):
        ones_rows = jnp.ones((N, D), jnp.float32)

        def pool(h_ref, seg_ref):
            seg = seg_ref[...]
            gids = lax.broadcasted_iota(jnp.int32, (N, NG), 1)
            onehot = (seg == gids).astype(jnp.float32)
            tot = lax.dot_general(onehot, h_ref[0], (((0,), (0,)), ((), ())),
                                  preferred_element_type=jnp.float32)
            cnt = lax.dot_general(onehot, ones_rows, (((0,), (0,)), ((), ())),
                                  preferred_element_type=jnp.float32)
            return tot / jnp.maximum(cnt, 1.0)

        z = jnp.concatenate([pool(hm_ref, mb_ref), pool(hp_ref, pb_ref)], axis=1)
        x = jnp.maximum(_dotT(z, w1[...]) + b1[...], 0.0)
        y = jnp.sum(x * w2[...], axis=1, keepdims=True) + b2[0, 0]
        o_ref[...] = 1.0 / (1.0 + jnp.exp(-y))

    full = lambda shp: pl.BlockSpec(shp, lambda i: tuple(0 for _ in shp))
    halfm = pl.BlockSpec((1, N, D), lambda i: (0, 0, 0))
    halfp = pl.BlockSpec((1, N, D), lambda i: (1, 0, 0))
    h3 = h_all.reshape(2, N, D)
    return pl.pallas_call(
        body,
        grid=(1,),
        in_specs=[halfm, halfp, full((N, 1)), full((N, 1)),
                  full((D, 2 * D)), full((1, D)), full((1, D)), full((1, 1))],
        out_specs=full((NG, 1)),
        out_shape=jax.ShapeDtypeStruct((NG, 1), jnp.float32),
    )(h3, h3, mb, pb, fc1w, fc1b, fc2w, fc2b)


def _stack(p, names):
    return jnp.stack([p[n] for n in names])


def kernel(x_mol, edge_index_mol, mol_batch, x_prot, edge_index_prot,
           prot_batch, params):
    p = params
    x_all = jnp.concatenate([x_mol, x_prot])
    src = jnp.concatenate([edge_index_mol[0], edge_index_prot[0] + N])
    dst = jnp.concatenate([edge_index_mol[1], edge_index_prot[1] + N])
    src2d = src.reshape(E2 // CHUNK, CHUNK)
    dst2d = dst.reshape(E2 // CHUNK, CHUNK)
    zeros = jnp.zeros((N2, D), jnp.float32)
    ones_cw = jnp.ones((CHUNK, CW), jnp.float32)

    cnt_parts = _make_sc_hist()(dst2d, zeros, ones_cw)

    wl1 = _stack(p, ['mol_Wl1', 'prot_Wl1'])
    bl1 = _stack(p, ['mol_bl1', 'prot_bl1']).reshape(2, 1, D)
    wr1 = _stack(p, ['mol_Wr1', 'prot_Wr1'])
    wl2 = _stack(p, ['mol_Wl2', 'prot_Wl2'])
    bl2 = _stack(p, ['mol_bl2', 'prot_bl2']).reshape(2, 1, D)
    wr2 = _stack(p, ['mol_Wr2', 'prot_Wr2'])
    wq = _stack(p, ['m2p_WQ', 'p2m_WQ'])
    bq = _stack(p, ['m2p_bQ', 'p2m_bQ']).reshape(2, 1, D)
    wk = _stack(p, ['p2m_WK', 'm2p_WK'])
    bk = _stack(p, ['p2m_bK', 'm2p_bK']).reshape(2, 1, D)
    wv = _stack(p, ['p2m_WV', 'm2p_WV'])
    bv = _stack(p, ['p2m_bV', 'm2p_bV']).reshape(2, 1, D)

    segsum = _make_sc_segsum()
    parts1 = segsum(x_all, src2d, dst2d, zeros)
    h1 = _sage1_tc(parts1, cnt_parts, x_all, wl1, bl1, wr1)
    parts2 = segsum(h1, src2d, dst2d, zeros)
    h2, qs, ks, vs = _sage2_proj_tc(parts2, cnt_parts, h1,
                                    wl2, bl2, wr2, wq, bq, wk, bk, wv, bv)

    h_c = _attn_tc(qs, ks, vs, h2)

    out = _pool_tc(h_c, mol_batch.reshape(N, 1), prot_batch.reshape(N, 1),
                   p['fc1_W'], p['fc1_b'].reshape(1, D),
                   p['fc2_W'].reshape(1, D), p['fc2_b'].reshape(1, 1))
    return out.reshape(NG)
